# baseline (speedup 1.0000x reference)
import functools

import jax
import jax.numpy as jnp
from jax import lax
from jax.experimental import pallas as pl
from jax.experimental.pallas import tpu as pltpu
from jax.experimental.pallas import tpu_sc as plsc

N = 100000
D_ATOM = 64
D_RES = 32
D_NUM = 16
D_OUT = D_ATOM + D_RES + D_NUM

NC = 2
NS = 16
NW = NC * NS

BPW = 3128
CHUNK = 448
NCHUNK = 7


def _make_kernel():
    mesh = plsc.VectorSubcoreMesh(core_axis_name="c", subcore_axis_name="s")

    @functools.partial(
        pl.kernel,
        mesh=mesh,
        out_type=jax.ShapeDtypeStruct((N, D_OUT), jnp.float32),
        compiler_params=pltpu.CompilerParams(use_tc_tiling_on_sc=False),
        scratch_types=[
            pltpu.VMEM((CHUNK,), jnp.int32),
            pltpu.VMEM((CHUNK,), jnp.int32),
            pltpu.VMEM((CHUNK, D_ATOM), jnp.float32),
            pltpu.VMEM((CHUNK, D_RES), jnp.float32),
            pltpu.VMEM((CHUNK, D_NUM), jnp.float32),
            pltpu.SemaphoreType.DMA,
            pltpu.SemaphoreType.DMA,
        ],
    )
    def emb_kernel(atom_idx, res_idx, charge, w_atom, w_res, out,
                   idxa_v, idxr_v, rows_a, rows_r, rows_c, sem_a, sem_r):
        wid = lax.axis_index("s") * NC + lax.axis_index("c")
        base = jnp.minimum(wid * BPW, N - BPW)

        def chunk_body(ci, carry):
            start = jnp.minimum(base + ci * CHUNK, base + (BPW - CHUNK))
            pltpu.sync_copy(atom_idx.at[pl.ds(start, CHUNK)], idxa_v)
            pltpu.sync_copy(res_idx.at[pl.ds(start, CHUNK)], idxr_v)
            cpa = pltpu.async_copy(w_atom.at[idxa_v], rows_a, sem_a)
            cpr = pltpu.async_copy(w_res.at[idxr_v], rows_r, sem_r)
            pltpu.sync_copy(charge.at[pl.ds(start, CHUNK)], rows_c)
            cpa.wait()
            cpr.wait()
            pltpu.sync_copy(rows_a, out.at[pl.ds(start, CHUNK), pl.ds(0, D_ATOM)])
            pltpu.sync_copy(rows_r, out.at[pl.ds(start, CHUNK), pl.ds(D_ATOM, D_RES)])
            pltpu.sync_copy(rows_c, out.at[pl.ds(start, CHUNK), pl.ds(D_ATOM + D_RES, D_NUM)])
            return carry

        lax.fori_loop(0, NCHUNK, chunk_body, 0)

    return emb_kernel


_EMB = _make_kernel()


def kernel(atom_type, residue_type, charge, W_atom, W_res):
    return _EMB(atom_type.reshape(-1).astype(jnp.int32),
                residue_type.reshape(-1).astype(jnp.int32),
                charge, W_atom, W_res)
